# Initial kernel scaffold; baseline (speedup 1.0000x reference)
import jax
import jax.numpy as jnp
from jax.experimental import pallas as pl


def kernel(xyz, params):
    raise NotImplementedError("write your pallas kernel here")


# speedup vs baseline: 2.9663x; 2.9663x over previous
import functools

import jax
import jax.numpy as jnp
from jax import lax
from jax.experimental import pallas as pl
from jax.experimental.pallas import tpu as pltpu
from jax.experimental.pallas import tpu_sc as plsc

_BIG = 3.0e38


def _knn_body(q_ref, r_ref, idx_ref, dk_ref, *, k, n_rows):
    q = q_ref[0]
    r = r_ref[0]
    mb = q.shape[0]
    d1 = lax.dot_general(q, r, (((1,), (1,)), ((), ())),
                         preferred_element_type=jnp.float32)
    r2c = jnp.sum(r * r, axis=1, keepdims=True)
    r2row = lax.dot_general(jnp.ones((mb, 1), jnp.float32), r2c,
                            (((1,), (1,)), ((), ())),
                            precision=lax.Precision.HIGHEST,
                            preferred_element_type=jnp.float32)
    q2 = jnp.sum(q * q, axis=1, keepdims=True)
    d = (q2 - 2.0 * d1) + r2row
    iota = lax.broadcasted_iota(jnp.int32, d.shape, 1)
    off = pl.program_id(0) * n_rows
    for j in range(k):
        m = jnp.min(d, axis=1, keepdims=True)
        am = jnp.min(jnp.where(d == m, iota, n_rows), axis=1, keepdims=True)
        idx_ref[0, :, j:j + 1] = am + off
        dk_ref[0, :, j:j + 1] = m
        d = jnp.where(iota == am, _BIG, d)


def _knn(q, r, k, mb):
    b, m, _ = q.shape
    n_rows = r.shape[1]
    grid = (b, m // mb)
    return pl.pallas_call(
        functools.partial(_knn_body, k=k, n_rows=n_rows),
        grid=grid,
        in_specs=[
            pl.BlockSpec((1, mb, 3), lambda bi, i: (bi, i, 0)),
            pl.BlockSpec((1, n_rows, 3), lambda bi, i: (bi, 0, 0)),
        ],
        out_specs=[
            pl.BlockSpec((1, mb, k), lambda bi, i: (bi, i, 0)),
            pl.BlockSpec((1, mb, k), lambda bi, i: (bi, i, 0)),
        ],
        out_shape=[
            jax.ShapeDtypeStruct((b, m, k), jnp.int32),
            jax.ShapeDtypeStruct((b, m, k), jnp.float32),
        ],
    )(q, r)


def _sc_gather(table, idx2d, d_cols):
    ch = 128
    n_total = idx2d.shape[0] * ch
    n_per_w = n_total // 32
    n_ch = n_per_w // ch
    mesh = plsc.VectorSubcoreMesh(core_axis_name="c", subcore_axis_name="s")

    @functools.partial(
        pl.kernel,
        mesh=mesh,
        compiler_params=pltpu.CompilerParams(use_tc_tiling_on_sc=False),
        out_type=jax.ShapeDtypeStruct((n_total, d_cols), jnp.float32),
        scratch_types=[
            pltpu.VMEM((n_ch, ch), jnp.int32),
            pltpu.VMEM((ch, d_cols), jnp.float32),
            pltpu.VMEM((ch, d_cols), jnp.float32),
            pltpu.SemaphoreType.DMA,
            pltpu.SemaphoreType.DMA,
        ],
    )
    def gk(table_hbm, idx_hbm, out_hbm, idx_v, buf0, buf1, sem0, sem1):
        wid = lax.axis_index("s") * 2 + lax.axis_index("c")
        base = wid * n_per_w
        pltpu.sync_copy(idx_hbm.at[pl.ds(wid * n_ch, n_ch)], idx_v)
        bufs = (buf0, buf1)
        sems = (sem0, sem1)
        pltpu.async_copy(table_hbm.at[idx_v.at[0]], buf0, sem0)

        def body(i, c):
            jj = i * 2
            for t in range(2):
                j = jj + t
                nb, ns = bufs[1 - t], sems[1 - t]

                @pl.when(j + 1 < n_ch)
                def _():
                    pltpu.async_copy(table_hbm.at[idx_v.at[j + 1]], nb, ns)

                pltpu.make_async_copy(
                    table_hbm.at[idx_v.at[j]], bufs[t], sems[t]).wait()
                pltpu.sync_copy(bufs[t], out_hbm.at[pl.ds(base + j * ch, ch)])
            return c

        lax.fori_loop(0, n_ch // 2, body, 0)

    return gk(table, idx2d)


def _sa_mlp_body(nbr_ref, q_ref, w0p_ref, b0_ref, w1_ref, b1_ref,
                 w2_ref, b2_ref, out_ref, *, mb, k, out_pad_q):
    nbr = nbr_ref[...]
    q = q_ref[0]
    dpad = nbr.shape[1]
    h0 = w0p_ref.shape[1]
    qpad = jnp.concatenate(
        [q, jnp.zeros((mb, dpad - 3), jnp.float32)], axis=1)
    g = nbr.reshape(mb, k, dpad) - qpad[:, None, :]
    l1 = jnp.dot(g.reshape(mb * k, dpad), w0p_ref[...],
                 preferred_element_type=jnp.float32)
    l1 = jnp.maximum(l1 + b0_ref[...], 0.0)
    l2 = jnp.maximum(
        jnp.dot(l1, w1_ref[...], preferred_element_type=jnp.float32)
        + b1_ref[...], 0.0)
    l3 = jnp.maximum(
        jnp.dot(l2, w2_ref[...], preferred_element_type=jnp.float32)
        + b2_ref[...], 0.0)
    c_out = l3.shape[1]
    f = jnp.max(l3.reshape(mb, k, c_out), axis=1)
    if out_pad_q:
        pad = out_ref.shape[1] - 3 - c_out
        out_ref[...] = jnp.concatenate(
            [q, f, jnp.zeros((mb, pad), jnp.float32)], axis=1)
    else:
        out_ref[...] = f


def _sa_mlp(nbr, q, w0p, b0, w1, b1, w2, b2, mb, k, out_cols, out_pad_q):
    b, m, _ = q.shape
    dpad = nbr.shape[1]
    grid = (b, m // mb)
    nblk = m // mb

    def wspec(w):
        return pl.BlockSpec(w.shape, lambda bi, i: tuple(0 for _ in w.shape))

    return pl.pallas_call(
        functools.partial(_sa_mlp_body, mb=mb, k=k, out_pad_q=out_pad_q),
        grid=grid,
        in_specs=[
            pl.BlockSpec((mb * k, dpad), lambda bi, i: (bi * nblk + i, 0)),
            pl.BlockSpec((1, mb, 3), lambda bi, i: (bi, i, 0)),
            wspec(w0p), wspec(b0), wspec(w1), wspec(b1),
            wspec(w2), wspec(b2),
        ],
        out_specs=pl.BlockSpec((mb, out_cols), lambda bi, i: (bi * nblk + i, 0)),
        out_shape=jax.ShapeDtypeStruct((b * m, out_cols), jnp.float32),
    )(nbr, q, w0p, b0, w1, b1, w2, b2)


def _sa3reg_body(x2_ref, f2_ref, w0a_ref, w0b_ref, b0_ref, w1_ref, b1_ref,
                 w2_ref, b2_ref, rw0_ref, rb0_ref, rw1_ref, rb1_ref, reg_ref):
    x2 = x2_ref[0]
    f2 = f2_ref[...]
    l1 = jnp.maximum(
        jnp.dot(x2, w0a_ref[...], preferred_element_type=jnp.float32)
        + jnp.dot(f2, w0b_ref[...], preferred_element_type=jnp.float32)
        + b0_ref[...], 0.0)
    l2 = jnp.maximum(
        jnp.dot(l1, w1_ref[...], preferred_element_type=jnp.float32)
        + b1_ref[...], 0.0)
    l3 = jnp.maximum(
        jnp.dot(l2, w2_ref[...], preferred_element_type=jnp.float32)
        + b2_ref[...], 0.0)
    btl = jnp.max(l3, axis=0, keepdims=True)
    r1 = jnp.maximum(
        jnp.dot(btl, rw0_ref[...], preferred_element_type=jnp.float32)
        + rb0_ref[...], 0.0)
    reg_ref[0] = (jnp.dot(r1, rw1_ref[...],
                          preferred_element_type=jnp.float32)
                  + rb1_ref[...])


def _sa3reg(x2, f2, weights):
    b = x2.shape[0]
    m2 = x2.shape[1]

    def wspec(w):
        return pl.BlockSpec(w.shape, lambda bi: tuple(0 for _ in w.shape))

    return pl.pallas_call(
        _sa3reg_body,
        grid=(b,),
        in_specs=[pl.BlockSpec((1, m2, 3), lambda bi: (bi, 0, 0)),
                  pl.BlockSpec((m2, 256), lambda bi: (bi, 0))]
        + [wspec(w) for w in weights],
        out_specs=pl.BlockSpec((1, 1, 3), lambda bi: (bi, 0, 0)),
        out_shape=jax.ShapeDtypeStruct((b, 1, 3), jnp.float32),
    )(x2, f2, *weights)


def _fp_body(g_ref, w_ref, skip_ref, w0a_ref, w0b_ref, b0_ref, w1_ref,
             b1_ref, ref8, *rest, mb, c_src, skip_3d):
    w = w_ref[0]
    g = g_ref[...].reshape(mb, 3, c_src)
    interp = jnp.sum(g * w[:, :, None], axis=1)
    skip = skip_ref[0] if skip_3d else skip_ref[...]
    l1 = jnp.maximum(
        jnp.dot(interp, w0a_ref[...], preferred_element_type=jnp.float32)
        + jnp.dot(skip, w0b_ref[...], preferred_element_type=jnp.float32)
        + b0_ref[...], 0.0)
    l2 = jnp.maximum(
        jnp.dot(l1, w1_ref[...], preferred_element_type=jnp.float32)
        + b1_ref[...], 0.0)
    if rest:
        sw0_ref = ref8
        sb0_ref, sw1_ref, sb1_ref, seg_ref = rest
        s1 = jnp.maximum(
            jnp.dot(l2, sw0_ref[...], preferred_element_type=jnp.float32)
            + sb0_ref[...], 0.0)
        seg_ref[0] = (jnp.dot(s1, sw1_ref[...],
                              preferred_element_type=jnp.float32)
                      + sb1_ref[...])
    else:
        ref8[...] = l2


def _fp_mlp(g, w, skip, weights, mb, c_src, c_out, skip_3d, seg_out):
    b, m, _ = w.shape
    grid = (b, m // mb)
    nblk = m // mb

    def wspec(w):
        return pl.BlockSpec(w.shape, lambda bi, i: tuple(0 for _ in w.shape))

    if skip_3d:
        skip_spec = pl.BlockSpec((1, mb, 3), lambda bi, i: (bi, i, 0))
    else:
        skip_spec = pl.BlockSpec((mb, skip.shape[1]),
                                 lambda bi, i: (bi * nblk + i, 0))
    if seg_out:
        out_spec = pl.BlockSpec((1, mb, 3), lambda bi, i: (bi, i, 0))
        out_shape = jax.ShapeDtypeStruct((b, m, 3), jnp.float32)
    else:
        out_spec = pl.BlockSpec((mb, c_out), lambda bi, i: (bi * nblk + i, 0))
        out_shape = jax.ShapeDtypeStruct((b * m, c_out), jnp.float32)

    return pl.pallas_call(
        functools.partial(_fp_body, mb=mb, c_src=c_src, skip_3d=skip_3d),
        grid=grid,
        in_specs=[
            pl.BlockSpec((mb * 3, c_src), lambda bi, i: (bi * nblk + i, 0)),
            pl.BlockSpec((1, mb, 3), lambda bi, i: (bi, i, 0)),
            skip_spec,
        ] + [wspec(w) for w in weights],
        out_specs=out_spec,
        out_shape=out_shape,
    )(g, w, skip, *weights)


def _row(v):
    return v.reshape(1, -1)


def _fp_knn_weights(q, r, nref):
    d = (jnp.sum(q * q, -1, keepdims=True)
         - 2.0 * jnp.einsum("bmd,bnd->bmn", q, r)
         + jnp.sum(r * r, -1)[:, None, :])
    neg, idx = jax.lax.top_k(-d, 3)
    dk = -neg
    w = 1.0 / (dk + 1e-8)
    w = w / jnp.sum(w, -1, keepdims=True)
    b = q.shape[0]
    gidx = idx + (jnp.arange(b, dtype=idx.dtype) * nref)[:, None, None]
    return gidx, w


def kernel(xyz, params):
    p = params
    b, n, _ = xyz.shape
    m1, m2, k = n // 4, n // 16, 32
    xyz1 = xyz[:, ::4, :]
    xyz2 = xyz1[:, ::4, :]

    idx1, _ = _knn(xyz1, xyz, k, mb=256)
    xyz_flat = xyz.reshape(b * n, 3)
    table1 = jnp.pad(jnp.concatenate([xyz_flat, xyz_flat], axis=1),
                     ((0, 0), (0, 10)))
    g1 = _sc_gather(table1, idx1.reshape(-1, 128), 16)
    w0p = jnp.pad(p["sa1_W0"], ((0, 10), (0, 0)))
    table2 = _sa_mlp(
        g1, xyz1, w0p, _row(p["sa1_b0"]), p["sa1_W1"],
        _row(p["sa1_b1"]), p["sa1_W2"], _row(p["sa1_b2"]),
        mb=256, k=k, out_cols=144, out_pad_q=True)

    idx2, _ = _knn(xyz2, xyz1, k, mb=256)
    g2 = _sc_gather(table2, idx2.reshape(-1, 128), 144)
    w0p = jnp.pad(p["sa2_W0"], ((0, 13), (0, 0)))
    f2 = _sa_mlp(
        g2, xyz2, w0p, _row(p["sa2_b0"]), p["sa2_W1"],
        _row(p["sa2_b1"]), p["sa2_W2"], _row(p["sa2_b2"]),
        mb=256, k=k, out_cols=256, out_pad_q=False)

    w0 = p["sa3_W0"]
    reg = _sa3reg(
        xyz2, f2,
        (w0[:3], w0[3:], _row(p["sa3_b0"]), p["sa3_W1"], _row(p["sa3_b1"]),
         p["sa3_W2"], _row(p["sa3_b2"]), p["reg_head_W0"],
         _row(p["reg_head_b0"]), p["reg_head_W1"], _row(p["reg_head_b1"])))
    reg = reg.reshape(b, 3)

    idxf2, wf2 = _fp_knn_weights(xyz1, xyz2, m2)
    gf2 = _sc_gather(f2, idxf2.reshape(-1, 128), 256)
    f1_flat = table2[:, 3:131]
    w0 = p["fp2_W0"]
    u1 = _fp_mlp(
        gf2, wf2, f1_flat,
        (w0[:256], w0[256:], _row(p["fp2_b0"]), p["fp2_W1"],
         _row(p["fp2_b1"])),
        mb=1024, c_src=256, c_out=128, skip_3d=False, seg_out=False)

    idxf1, wf1 = _fp_knn_weights(xyz, xyz1, m1)
    gf1 = _sc_gather(u1, idxf1.reshape(-1, 128), 128)
    w0 = p["fp1_W0"]
    seg = _fp_mlp(
        gf1, wf1, xyz,
        (w0[:128], w0[128:], _row(p["fp1_b0"]), p["fp1_W1"],
         _row(p["fp1_b1"]), p["seg_head_W0"], _row(p["seg_head_b0"]),
         p["seg_head_W1"], _row(p["seg_head_b1"])),
        mb=1024, c_src=128, c_out=128, skip_3d=True, seg_out=True)

    return seg, reg


# speedup vs baseline: 3.1775x; 1.0712x over previous
import functools

import jax
import jax.numpy as jnp
from jax import lax
from jax.experimental import pallas as pl
from jax.experimental.pallas import tpu as pltpu
from jax.experimental.pallas import tpu_sc as plsc

_BIG = 3.0e38


def _knn_body(q_ref, r_ref, idx_ref, *, k, n_rows):
    q = q_ref[0]
    r = r_ref[0]
    mb = q.shape[0]
    d1 = lax.dot_general(q, r, (((1,), (1,)), ((), ())),
                         preferred_element_type=jnp.float32)
    r2c = jnp.sum(r * r, axis=1, keepdims=True)
    r2row = lax.dot_general(jnp.ones((mb, 1), jnp.float32), r2c,
                            (((1,), (1,)), ((), ())),
                            precision=lax.Precision.HIGHEST,
                            preferred_element_type=jnp.float32)
    q2 = jnp.sum(q * q, axis=1, keepdims=True)
    d = (q2 - 2.0 * d1) + r2row
    iota = lax.broadcasted_iota(jnp.int32, d.shape, 1)
    off = pl.program_id(0) * n_rows
    for j in range(k):
        am = jnp.argmin(d, axis=1).astype(jnp.int32)[:, None]
        idx_ref[0, :, j:j + 1] = am + off
        d = jnp.where(iota == am, _BIG, d)


def _knn(q, r, k, mb):
    b, m, _ = q.shape
    n_rows = r.shape[1]
    grid = (b, m // mb)
    return pl.pallas_call(
        functools.partial(_knn_body, k=k, n_rows=n_rows),
        grid=grid,
        in_specs=[
            pl.BlockSpec((1, mb, 3), lambda bi, i: (bi, i, 0)),
            pl.BlockSpec((1, n_rows, 3), lambda bi, i: (bi, 0, 0)),
        ],
        out_specs=pl.BlockSpec((1, mb, k), lambda bi, i: (bi, i, 0)),
        out_shape=jax.ShapeDtypeStruct((b, m, k), jnp.int32),
    )(q, r)


def _sc_gather(table, idx2d, d_cols):
    ch = 128
    n_total = idx2d.shape[0] * ch
    n_per_w = n_total // 32
    n_ch = n_per_w // ch
    mesh = plsc.VectorSubcoreMesh(core_axis_name="c", subcore_axis_name="s")

    @functools.partial(
        pl.kernel,
        mesh=mesh,
        compiler_params=pltpu.CompilerParams(use_tc_tiling_on_sc=False),
        out_type=jax.ShapeDtypeStruct((n_total, d_cols), jnp.float32),
        scratch_types=[
            pltpu.VMEM((n_ch, ch), jnp.int32),
            pltpu.VMEM((ch, d_cols), jnp.float32),
            pltpu.VMEM((ch, d_cols), jnp.float32),
            pltpu.SemaphoreType.DMA,
            pltpu.SemaphoreType.DMA,
        ],
    )
    def gk(table_hbm, idx_hbm, out_hbm, idx_v, buf0, buf1, sem0, sem1):
        wid = lax.axis_index("s") * 2 + lax.axis_index("c")
        base = wid * n_per_w
        pltpu.sync_copy(idx_hbm.at[pl.ds(wid * n_ch, n_ch)], idx_v)
        bufs = (buf0, buf1)
        sems = (sem0, sem1)
        pltpu.async_copy(table_hbm.at[idx_v.at[0]], buf0, sem0)

        def body(i, c):
            jj = i * 2
            for t in range(2):
                j = jj + t
                nb, ns = bufs[1 - t], sems[1 - t]

                @pl.when(j + 1 < n_ch)
                def _():
                    pltpu.async_copy(table_hbm.at[idx_v.at[j + 1]], nb, ns)

                pltpu.make_async_copy(
                    table_hbm.at[idx_v.at[j]], bufs[t], sems[t]).wait()
                pltpu.sync_copy(bufs[t], out_hbm.at[pl.ds(base + j * ch, ch)])
            return c

        lax.fori_loop(0, n_ch // 2, body, 0)

    return gk(table, idx2d)


def _sa_mlp_body(nbr_ref, q_ref, w0p_ref, b0_ref, w1_ref, b1_ref,
                 w2_ref, b2_ref, out_ref, *, mb, k, out_pad_q):
    nbr = nbr_ref[...]
    q = q_ref[0]
    dpad = nbr.shape[1]
    h0 = w0p_ref.shape[1]
    qpad = jnp.concatenate(
        [q, jnp.zeros((mb, dpad - 3), jnp.float32)], axis=1)
    g = nbr.reshape(mb, k, dpad) - qpad[:, None, :]
    l1 = jnp.dot(g.reshape(mb * k, dpad), w0p_ref[...],
                 preferred_element_type=jnp.float32)
    l1 = jnp.maximum(l1 + b0_ref[...], 0.0)
    l2 = jnp.maximum(
        jnp.dot(l1, w1_ref[...], preferred_element_type=jnp.float32)
        + b1_ref[...], 0.0)
    l3 = jnp.maximum(
        jnp.dot(l2, w2_ref[...], preferred_element_type=jnp.float32)
        + b2_ref[...], 0.0)
    c_out = l3.shape[1]
    f = jnp.max(l3.reshape(mb, k, c_out), axis=1)
    if out_pad_q:
        pad = out_ref.shape[1] - 3 - c_out
        out_ref[...] = jnp.concatenate(
            [q, f, jnp.zeros((mb, pad), jnp.float32)], axis=1)
    else:
        out_ref[...] = f


def _sa_mlp(nbr, q, w0p, b0, w1, b1, w2, b2, mb, k, out_cols, out_pad_q):
    b, m, _ = q.shape
    dpad = nbr.shape[1]
    grid = (b, m // mb)
    nblk = m // mb

    def wspec(w):
        return pl.BlockSpec(w.shape, lambda bi, i: tuple(0 for _ in w.shape))

    return pl.pallas_call(
        functools.partial(_sa_mlp_body, mb=mb, k=k, out_pad_q=out_pad_q),
        grid=grid,
        in_specs=[
            pl.BlockSpec((mb * k, dpad), lambda bi, i: (bi * nblk + i, 0)),
            pl.BlockSpec((1, mb, 3), lambda bi, i: (bi, i, 0)),
            wspec(w0p), wspec(b0), wspec(w1), wspec(b1),
            wspec(w2), wspec(b2),
        ],
        out_specs=pl.BlockSpec((mb, out_cols), lambda bi, i: (bi * nblk + i, 0)),
        out_shape=jax.ShapeDtypeStruct((b * m, out_cols), jnp.float32),
    )(nbr, q, w0p, b0, w1, b1, w2, b2)


def _sa3reg_body(x2_ref, f2_ref, w0a_ref, w0b_ref, b0_ref, w1_ref, b1_ref,
                 w2_ref, b2_ref, rw0_ref, rb0_ref, rw1_ref, rb1_ref, reg_ref):
    x2 = x2_ref[0]
    f2 = f2_ref[...]
    l1 = jnp.maximum(
        jnp.dot(x2, w0a_ref[...], preferred_element_type=jnp.float32)
        + jnp.dot(f2, w0b_ref[...], preferred_element_type=jnp.float32)
        + b0_ref[...], 0.0)
    l2 = jnp.maximum(
        jnp.dot(l1, w1_ref[...], preferred_element_type=jnp.float32)
        + b1_ref[...], 0.0)
    l3 = jnp.maximum(
        jnp.dot(l2, w2_ref[...], preferred_element_type=jnp.float32)
        + b2_ref[...], 0.0)
    btl = jnp.max(l3, axis=0, keepdims=True)
    r1 = jnp.maximum(
        jnp.dot(btl, rw0_ref[...], preferred_element_type=jnp.float32)
        + rb0_ref[...], 0.0)
    reg_ref[0] = (jnp.dot(r1, rw1_ref[...],
                          preferred_element_type=jnp.float32)
                  + rb1_ref[...])


def _sa3reg(x2, f2, weights):
    b = x2.shape[0]
    m2 = x2.shape[1]

    def wspec(w):
        return pl.BlockSpec(w.shape, lambda bi: tuple(0 for _ in w.shape))

    return pl.pallas_call(
        _sa3reg_body,
        grid=(b,),
        in_specs=[pl.BlockSpec((1, m2, 3), lambda bi: (bi, 0, 0)),
                  pl.BlockSpec((m2, 256), lambda bi: (bi, 0))]
        + [wspec(w) for w in weights],
        out_specs=pl.BlockSpec((1, 1, 3), lambda bi: (bi, 0, 0)),
        out_shape=jax.ShapeDtypeStruct((b, 1, 3), jnp.float32),
    )(x2, f2, *weights)


def _fp_body(g_ref, w_ref, skip_ref, w0a_ref, w0b_ref, b0_ref, w1_ref,
             b1_ref, ref8, *rest, mb, c_src, skip_3d):
    w = w_ref[0]
    g = g_ref[...].reshape(mb, 3, c_src)
    interp = jnp.sum(g * w[:, :, None], axis=1)
    skip = skip_ref[0] if skip_3d else skip_ref[...]
    l1 = jnp.maximum(
        jnp.dot(interp, w0a_ref[...], preferred_element_type=jnp.float32)
        + jnp.dot(skip, w0b_ref[...], preferred_element_type=jnp.float32)
        + b0_ref[...], 0.0)
    l2 = jnp.maximum(
        jnp.dot(l1, w1_ref[...], preferred_element_type=jnp.float32)
        + b1_ref[...], 0.0)
    if rest:
        sw0_ref = ref8
        sb0_ref, sw1_ref, sb1_ref, seg_ref = rest
        s1 = jnp.maximum(
            jnp.dot(l2, sw0_ref[...], preferred_element_type=jnp.float32)
            + sb0_ref[...], 0.0)
        seg_ref[0] = (jnp.dot(s1, sw1_ref[...],
                              preferred_element_type=jnp.float32)
                      + sb1_ref[...])
    else:
        ref8[...] = l2


def _fp_mlp(g, w, skip, weights, mb, c_src, c_out, skip_3d, seg_out):
    b, m, _ = w.shape
    grid = (b, m // mb)
    nblk = m // mb

    def wspec(w):
        return pl.BlockSpec(w.shape, lambda bi, i: tuple(0 for _ in w.shape))

    if skip_3d:
        skip_spec = pl.BlockSpec((1, mb, 3), lambda bi, i: (bi, i, 0))
    else:
        skip_spec = pl.BlockSpec((mb, skip.shape[1]),
                                 lambda bi, i: (bi * nblk + i, 0))
    if seg_out:
        out_spec = pl.BlockSpec((1, mb, 3), lambda bi, i: (bi, i, 0))
        out_shape = jax.ShapeDtypeStruct((b, m, 3), jnp.float32)
    else:
        out_spec = pl.BlockSpec((mb, c_out), lambda bi, i: (bi * nblk + i, 0))
        out_shape = jax.ShapeDtypeStruct((b * m, c_out), jnp.float32)

    return pl.pallas_call(
        functools.partial(_fp_body, mb=mb, c_src=c_src, skip_3d=skip_3d),
        grid=grid,
        in_specs=[
            pl.BlockSpec((mb * 3, c_src), lambda bi, i: (bi * nblk + i, 0)),
            pl.BlockSpec((1, mb, 3), lambda bi, i: (bi, i, 0)),
            skip_spec,
        ] + [wspec(w) for w in weights],
        out_specs=out_spec,
        out_shape=out_shape,
    )(g, w, skip, *weights)


def _row(v):
    return v.reshape(1, -1)


def _fp_knn_weights(q, r, nref):
    d = (jnp.sum(q * q, -1, keepdims=True)
         - 2.0 * jnp.einsum("bmd,bnd->bmn", q, r)
         + jnp.sum(r * r, -1)[:, None, :])
    neg, idx = jax.lax.top_k(-d, 3)
    dk = -neg
    w = 1.0 / (dk + 1e-8)
    w = w / jnp.sum(w, -1, keepdims=True)
    b = q.shape[0]
    gidx = idx + (jnp.arange(b, dtype=idx.dtype) * nref)[:, None, None]
    return gidx, w


def kernel(xyz, params):
    p = params
    b, n, _ = xyz.shape
    m1, m2, k = n // 4, n // 16, 32
    xyz1 = xyz[:, ::4, :]
    xyz2 = xyz1[:, ::4, :]

    idx1 = _knn(xyz1, xyz, k, mb=256)
    xyz_flat = xyz.reshape(b * n, 3)
    table1 = jnp.pad(jnp.concatenate([xyz_flat, xyz_flat], axis=1),
                     ((0, 0), (0, 10)))
    g1 = _sc_gather(table1, idx1.reshape(-1, 128), 16)
    w0p = jnp.pad(p["sa1_W0"], ((0, 10), (0, 0)))
    table2 = _sa_mlp(
        g1, xyz1, w0p, _row(p["sa1_b0"]), p["sa1_W1"],
        _row(p["sa1_b1"]), p["sa1_W2"], _row(p["sa1_b2"]),
        mb=256, k=k, out_cols=144, out_pad_q=True)

    idx2 = _knn(xyz2, xyz1, k, mb=256)
    g2 = _sc_gather(table2, idx2.reshape(-1, 128), 144)
    w0p = jnp.pad(p["sa2_W0"], ((0, 13), (0, 0)))
    f2 = _sa_mlp(
        g2, xyz2, w0p, _row(p["sa2_b0"]), p["sa2_W1"],
        _row(p["sa2_b1"]), p["sa2_W2"], _row(p["sa2_b2"]),
        mb=256, k=k, out_cols=256, out_pad_q=False)

    w0 = p["sa3_W0"]
    reg = _sa3reg(
        xyz2, f2,
        (w0[:3], w0[3:], _row(p["sa3_b0"]), p["sa3_W1"], _row(p["sa3_b1"]),
         p["sa3_W2"], _row(p["sa3_b2"]), p["reg_head_W0"],
         _row(p["reg_head_b0"]), p["reg_head_W1"], _row(p["reg_head_b1"])))
    reg = reg.reshape(b, 3)

    idxf2, wf2 = _fp_knn_weights(xyz1, xyz2, m2)
    gf2 = _sc_gather(f2, idxf2.reshape(-1, 128), 256)
    f1_flat = table2[:, 3:131]
    w0 = p["fp2_W0"]
    u1 = _fp_mlp(
        gf2, wf2, f1_flat,
        (w0[:256], w0[256:], _row(p["fp2_b0"]), p["fp2_W1"],
         _row(p["fp2_b1"])),
        mb=1024, c_src=256, c_out=128, skip_3d=False, seg_out=False)

    idxf1, wf1 = _fp_knn_weights(xyz, xyz1, m1)
    gf1 = _sc_gather(u1, idxf1.reshape(-1, 128), 128)
    w0 = p["fp1_W0"]
    seg = _fp_mlp(
        gf1, wf1, xyz,
        (w0[:128], w0[128:], _row(p["fp1_b0"]), p["fp1_W1"],
         _row(p["fp1_b1"]), p["seg_head_W0"], _row(p["seg_head_b0"]),
         p["seg_head_W1"], _row(p["seg_head_b1"])),
        mb=1024, c_src=128, c_out=128, skip_3d=True, seg_out=True)

    return seg, reg
